# Initial kernel scaffold; baseline (speedup 1.0000x reference)
import jax
import jax.numpy as jnp
from jax.experimental import pallas as pl


def kernel(point_feat, point_xyz, supernode_xyz, neighbor_idx, neighbor_mask, supernode_init_feat, W1, b1, W2, b2, ln_w, ln_b):
    raise NotImplementedError("write your pallas kernel here")


# speedup vs baseline: 7.7126x; 7.7126x over previous
import functools

import jax
import jax.numpy as jnp
from jax import lax
from jax.experimental import pallas as pl
from jax.experimental.pallas import tpu as pltpu
from jax.experimental.pallas import tpu_sc as plsc

D = 128
XP = 16
CH = 128
BM = 128


@functools.partial(jax.jit, static_argnums=(3, 4))
def _sc_gather(feat, xyzp, idx3, mk, n):
    info = plsc.get_sparse_core_info()
    nc, ns = info.num_cores, info.num_subcores
    nw = nc * ns
    per_w = mk // nw
    n_ch = per_w // CH

    mesh = plsc.VectorSubcoreMesh(core_axis_name="c", subcore_axis_name="s")

    @functools.partial(
        pl.kernel,
        mesh=mesh,
        compiler_params=pltpu.CompilerParams(use_tc_tiling_on_sc=False),
        out_type=(
            jax.ShapeDtypeStruct((mk, D), jnp.float32),
            jax.ShapeDtypeStruct((mk, XP), jnp.float32),
        ),
        scratch_types=[
            pltpu.VMEM((n_ch, CH), jnp.int32),
            pltpu.VMEM((CH, D), jnp.float32),
            pltpu.VMEM((CH, D), jnp.float32),
            pltpu.VMEM((CH, XP), jnp.float32),
            pltpu.VMEM((CH, XP), jnp.float32),
            pltpu.SemaphoreType.DMA,
            pltpu.SemaphoreType.DMA,
            pltpu.SemaphoreType.DMA,
            pltpu.SemaphoreType.DMA,
        ],
    )
    def gather_k(feat_hbm, xyzp_hbm, idx_hbm, outf_hbm, outx_hbm,
                 idx_v, f0, f1, x0, x1, sf0, sf1, sx0, sx1):
        wid = lax.axis_index("s") * nc + lax.axis_index("c")
        base = wid * per_w
        pltpu.sync_copy(idx_hbm.at[wid], idx_v)

        fbufs = (f0, f1)
        xbufs = (x0, x1)
        fsems = (sf0, sf1)
        xsems = (sx0, sx1)

        pend = {}

        def start(c):
            p = c % 2
            hf = pltpu.async_copy(feat_hbm.at[idx_v.at[c]], fbufs[p], fsems[p])
            hx = pltpu.async_copy(xyzp_hbm.at[idx_v.at[c]], xbufs[p], xsems[p])
            pend[c] = (hf, hx)

        start(0)
        for c in range(n_ch):
            if c + 1 < n_ch:
                start(c + 1)
            hf, hx = pend.pop(c)
            hf.wait()
            hx.wait()
            p = c % 2
            row = base + c * CH
            pltpu.sync_copy(fbufs[p], outf_hbm.at[pl.ds(row, CH)])
            pltpu.sync_copy(xbufs[p], outx_hbm.at[pl.ds(row, CH)])

    return gather_k(feat, xyzp, idx3)


def _tc_body(gf_ref, gx_ref, sup_ref, mask_ref, init_ref,
             w1f_ref, w1g_ref, b1_ref, w2_ref, b2_ref, lnw_ref, lnb_ref,
             out_ref, *, bm, k):
    ef = gf_ref[...]
    gx = gx_ref[...].reshape(bm, k, XP)
    sup = sup_ref[...]
    rel = gx - sup[:, None, :]
    d2 = jnp.sum(rel * rel, axis=-1, keepdims=True)
    dist = jnp.sqrt(d2)
    lane = lax.broadcasted_iota(jnp.int32, (bm, k, XP), 2)
    g = jnp.where(lane == 3, dist, rel)
    g2 = g.reshape(bm * k, XP)

    h = jnp.dot(ef, w1f_ref[...], preferred_element_type=jnp.float32)
    h = h + jnp.dot(g2, w1g_ref[...], preferred_element_type=jnp.float32)
    h = h + b1_ref[...]
    h = h * jax.nn.sigmoid(h)
    msg = jnp.dot(h, w2_ref[...], preferred_element_type=jnp.float32)
    msg = msg + b2_ref[...]

    mf = mask_ref[...]
    msg3 = msg.reshape(bm, k, D) * mf[:, :, None]
    s = jnp.sum(msg3, axis=1)
    denom = jnp.maximum(jnp.sum(mf, axis=1, keepdims=True), 1.0)
    x = init_ref[...] + s / denom

    mu = jnp.mean(x, axis=-1, keepdims=True)
    var = jnp.mean((x - mu) ** 2, axis=-1, keepdims=True)
    out_ref[...] = ((x - mu) * lax.rsqrt(var + 1e-5)) * lnw_ref[...] + lnb_ref[...]


def _tc_stage(gf, gx, supx, maskf, init, w1f, w1g, b1, w2, b2, lnw, lnb,
              bm, k, interpret=False):
    m = init.shape[0]
    grid = (m // bm,)
    full = lambda i: (0, 0)
    return pl.pallas_call(
        functools.partial(_tc_body, bm=bm, k=k),
        grid=grid,
        in_specs=[
            pl.BlockSpec((bm * k, D), lambda i: (i, 0)),
            pl.BlockSpec((bm * k, XP), lambda i: (i, 0)),
            pl.BlockSpec((bm, XP), lambda i: (i, 0)),
            pl.BlockSpec((bm, k), lambda i: (i, 0)),
            pl.BlockSpec((bm, D), lambda i: (i, 0)),
            pl.BlockSpec((D, D), full),
            pl.BlockSpec((XP, D), full),
            pl.BlockSpec((1, D), full),
            pl.BlockSpec((D, D), full),
            pl.BlockSpec((1, D), full),
            pl.BlockSpec((1, D), full),
            pl.BlockSpec((1, D), full),
        ],
        out_specs=pl.BlockSpec((bm, D), lambda i: (i, 0)),
        out_shape=jax.ShapeDtypeStruct((m, D), jnp.float32),
        interpret=interpret,
    )(gf, gx, supx, maskf, init, w1f, w1g, b1, w2, b2, lnw, lnb)


def kernel(point_feat, point_xyz, supernode_xyz, neighbor_idx, neighbor_mask,
           supernode_init_feat, W1, b1, W2, b2, ln_w, ln_b):
    b, n, d = point_feat.shape
    _, m, k = neighbor_idx.shape
    mk = m * k

    info = plsc.get_sparse_core_info()
    nw = info.num_cores * info.num_subcores
    n_ch = mk // (nw * CH)

    pf = point_feat[0]
    xyzp = jnp.pad(point_xyz[0], ((0, 0), (0, XP - 3)))
    idx3 = neighbor_idx[0].reshape(nw, n_ch, CH)

    gf, gx = _sc_gather(pf, xyzp, idx3, mk, n)

    supx = jnp.pad(supernode_xyz[0], ((0, 0), (0, XP - 3)))
    maskf = neighbor_mask[0].astype(jnp.float32)
    w1f = W1[:, :d].T
    w1g = jnp.pad(W1[:, d:d + 4].T, ((0, XP - 4), (0, 0)))
    out = _tc_stage(gf, gx, supx, maskf, supernode_init_feat[0],
                    w1f, w1g, b1.reshape(1, D), W2.T, b2.reshape(1, D),
                    ln_w.reshape(1, D), ln_b.reshape(1, D), BM, k)
    return out[None]
